# Initial kernel scaffold; baseline (speedup 1.0000x reference)
import jax
import jax.numpy as jnp
from jax.experimental import pallas as pl


def kernel(imgs, w1, b1, g1, be1, w2, b2, g2, be2, w3, b3, g3, be3, w4, b4, g4, be4, w5, b5, Wg, bg, Wb, Wa, ba):
    raise NotImplementedError("write your pallas kernel here")


# speedup vs baseline: 541.3642x; 541.3642x over previous
import functools

import jax
import jax.numpy as jnp
from jax import lax
from jax.experimental import pallas as pl
from jax.experimental.pallas import tpu as pltpu
from jax.experimental.pallas import tpu_sc as plsc

V = 33
V2 = V * V
V3 = V * V * V
LUT_LEN = 3 * V3
LUT_PAD = LUT_LEN + 5
NC, NS, L = 2, 16, 16
NW = NC * NS
CHUNK = 1024


def _conv2(x, w, b, stride):
    y = lax.conv_general_dilated(x, w, (stride, stride), ((1, 1), (1, 1)),
                                 dimension_numbers=('NCHW', 'OIHW', 'NCHW'))
    return y + b[None, :, None, None]


def _inorm2(x, g, b, eps=1e-5):
    m = jnp.mean(x, axis=(2, 3), keepdims=True)
    v = jnp.var(x, axis=(2, 3), keepdims=True)
    return (x - m) / jnp.sqrt(v + eps) * g[None, :, None, None] + b[None, :, None, None]


def _lrelu2(x):
    return jnp.where(x >= 0, x, 0.2 * x)


@functools.cache
def _build_transform(B, HW):
    half_px = HW // 2
    n_chunks = half_px // CHUNK
    mesh = plsc.VectorSubcoreMesh(core_axis_name="c", subcore_axis_name="s",
                                  num_cores=NC, num_subcores=NS)

    @functools.partial(
        pl.kernel,
        out_type=jax.ShapeDtypeStruct((B * 3 * HW,), jnp.float32),
        mesh=mesh,
        compiler_params=pltpu.CompilerParams(needs_layout_passes=False),
        scratch_types=[
            pltpu.VMEM((LUT_PAD,), jnp.float32),
            pltpu.VMEM((128,), jnp.float32),
            [pltpu.VMEM((CHUNK,), jnp.float32) for _ in range(3)],
            [pltpu.VMEM((CHUNK,), jnp.float32) for _ in range(3)],
        ],
    )
    def transform(imgs_hbm, luts_hbm, verts_hbm, out_hbm, lut_v, verts_v, in_v, out_v):
        wid = lax.axis_index("s") * NC + lax.axis_index("c")
        b = wid // 2
        base_px = (wid % 2) * half_px
        pltpu.sync_copy(luts_hbm.at[pl.ds(b * LUT_PAD, LUT_PAD)], lut_v)
        pltpu.sync_copy(verts_hbm.at[pl.ds(b * 128, 128)], verts_v)

        def do_chunk(g, carry):
            p0 = base_px + g * CHUNK
            for c in range(3):
                pltpu.sync_copy(
                    imgs_hbm.at[pl.ds((b * 3 + c) * HW + p0, CHUNK)], in_v[c])

            def do_vec(i, carry2):
                off = i * L
                idxs = []
                fracs = []
                for c in range(3):
                    val = in_v[c][pl.ds(off, L)]
                    cbase = jnp.full((L,), c * V, jnp.int32)
                    lo = jnp.zeros((L,), jnp.int32)
                    for s in (16, 8, 4, 2, 1):
                        probe = lo + s
                        vv = plsc.load_gather(verts_v, [probe + cbase])
                        lo = jnp.where(vv <= val, probe, lo)
                    lov = plsc.load_gather(verts_v, [lo + cbase])
                    hiv = plsc.load_gather(verts_v, [lo + cbase + 1])
                    fr = jnp.clip((val - lov) / jnp.maximum(hiv - lov, 1e-10),
                                  0.0, 1.0)
                    idxs.append(lo)
                    fracs.append(fr)
                ir, ig, ib = idxs
                fr, fg, fb = fracs
                base = ib * V2 + ig * V + ir
                wr0 = 1.0 - fr
                wg0 = 1.0 - fg
                wb0 = 1.0 - fb
                wbg00 = wb0 * wg0
                wbg01 = wb0 * fg
                wbg10 = fb * wg0
                wbg11 = fb * fg
                ws = (wbg00 * wr0, wbg00 * fr, wbg01 * wr0, wbg01 * fr,
                      wbg10 * wr0, wbg10 * fr, wbg11 * wr0, wbg11 * fr)
                offs = (0, 1, V, V + 1, V2, V2 + 1, V2 + V, V2 + V + 1)
                for c in range(3):
                    basec = base + c * V3
                    acc = ws[0] * plsc.load_gather(lut_v, [basec])
                    for k in range(1, 8):
                        acc = acc + ws[k] * plsc.load_gather(lut_v, [basec + offs[k]])
                    out_v[c][pl.ds(off, L)] = acc
                return carry2

            lax.fori_loop(0, CHUNK // L, do_vec, 0, unroll=False)
            for c in range(3):
                pltpu.sync_copy(
                    out_v[c], out_hbm.at[pl.ds((b * 3 + c) * HW + p0, CHUNK)])
            return carry

        lax.fori_loop(0, n_chunks, do_chunk, 0, unroll=False)

    return transform


def kernel(imgs, w1, b1, g1, be1, w2, b2, g2, be2, w3, b3, g3, be3,
           w4, b4, g4, be4, w5, b5, Wg, bg, Wb, Wa, ba):
    B, _, H, W = imgs.shape
    HW = H * W
    x = jax.image.resize(imgs, (B, 3, 256, 256), method='bilinear')
    x = _inorm2(_lrelu2(_conv2(x, w1, b1, 2)), g1, be1)
    x = _inorm2(_lrelu2(_conv2(x, w2, b2, 2)), g2, be2)
    x = _inorm2(_lrelu2(_conv2(x, w3, b3, 2)), g3, be3)
    x = _inorm2(_lrelu2(_conv2(x, w4, b4, 2)), g4, be4)
    x = _lrelu2(_conv2(x, w5, b5, 2))
    x = x.reshape(B, 128, 2, 4, 2, 4).mean(axis=(3, 5))
    codes = x.reshape(B, -1)
    weights = codes @ Wg.T + bg
    luts = weights @ Wb.T
    luts = jnp.pad(luts, ((0, 0), (0, LUT_PAD - LUT_LEN)))
    intervals = (codes @ Wa.T + ba).reshape(B, 3, V - 1)
    intervals = jax.nn.softmax(intervals, axis=-1)
    vertices = jnp.pad(jnp.cumsum(intervals, axis=-1), ((0, 0), (0, 0), (1, 0)))
    verts = jnp.pad(vertices.reshape(B, 3 * V), ((0, 0), (0, 128 - 3 * V)))

    out = _build_transform(B, HW)(imgs.reshape(-1), luts.reshape(-1),
                                  verts.reshape(-1))
    return out.reshape(B, 3, H, W)


# speedup vs baseline: 577.4303x; 1.0666x over previous
import functools

import jax
import jax.numpy as jnp
from jax import lax
from jax.experimental import pallas as pl
from jax.experimental.pallas import tpu as pltpu
from jax.experimental.pallas import tpu_sc as plsc

V = 33
V2 = V * V
V3 = V * V * V
LUT_LEN = 3 * V3
LUT_PAD = LUT_LEN + 5
NC, NS, L = 2, 16, 16
NW = NC * NS
CHUNK = 1024


def _conv2(x, w, b, stride):
    y = lax.conv_general_dilated(x, w, (stride, stride), ((1, 1), (1, 1)),
                                 dimension_numbers=('NCHW', 'OIHW', 'NCHW'))
    return y + b[None, :, None, None]


def _inorm2(x, g, b, eps=1e-5):
    m = jnp.mean(x, axis=(2, 3), keepdims=True)
    v = jnp.var(x, axis=(2, 3), keepdims=True)
    return (x - m) / jnp.sqrt(v + eps) * g[None, :, None, None] + b[None, :, None, None]


def _lrelu2(x):
    return jnp.where(x >= 0, x, 0.2 * x)


_NSPLIT = 8


@functools.cache
def _build_interleave(B, H, W):
    HW = H * W
    rpc = CHUNK // W
    rows = H // _NSPLIT
    ckb = rows // rpc
    FB = 3 * rows * W

    def body(x_ref, o_ref):
        for c in range(3):
            for k in range(ckb):
                for r in range(rpc):
                    o_ref[pl.ds(k * 3 * CHUNK + c * CHUNK + r * W, W)] = (
                        x_ref[0, c, k * rpc + r, :])

    return pl.pallas_call(
        body,
        grid=(B, _NSPLIT),
        in_specs=[pl.BlockSpec((1, 3, rows, W), lambda b, s: (b, 0, s, 0))],
        out_specs=pl.BlockSpec((FB,), lambda b, s: (b * _NSPLIT + s,)),
        out_shape=jax.ShapeDtypeStruct((B * 3 * HW,), jnp.float32),
    )


@functools.cache
def _build_deinterleave(B, H, W):
    HW = H * W
    rpc = CHUNK // W
    rows = H // _NSPLIT
    ckb = rows // rpc
    FB = 3 * rows * W

    def body(x_ref, o_ref):
        for c in range(3):
            for k in range(ckb):
                for r in range(rpc):
                    o_ref[0, c, k * rpc + r, :] = (
                        x_ref[pl.ds(k * 3 * CHUNK + c * CHUNK + r * W, W)])

    return pl.pallas_call(
        body,
        grid=(B, _NSPLIT),
        in_specs=[pl.BlockSpec((FB,), lambda b, s: (b * _NSPLIT + s,))],
        out_specs=pl.BlockSpec((1, 3, rows, W), lambda b, s: (b, 0, s, 0)),
        out_shape=jax.ShapeDtypeStruct((B, 3, H, W), jnp.float32),
    )


@functools.cache
def _build_transform(B, HW):
    half_px = HW // 2
    n_chunks = half_px // CHUNK
    n_chunks_s = HW // CHUNK
    CF = 3 * CHUNK
    mesh = plsc.VectorSubcoreMesh(core_axis_name="c", subcore_axis_name="s",
                                  num_cores=NC, num_subcores=NS)

    @functools.partial(
        pl.kernel,
        out_type=jax.ShapeDtypeStruct((B * 3 * HW,), jnp.float32),
        mesh=mesh,
        compiler_params=pltpu.CompilerParams(needs_layout_passes=False),
        scratch_types=[
            pltpu.VMEM((LUT_PAD,), jnp.float32),
            pltpu.VMEM((128,), jnp.float32),
            [pltpu.VMEM((CF,), jnp.float32) for _ in range(2)],
            [pltpu.VMEM((CF,), jnp.float32) for _ in range(2)],
            [pltpu.SemaphoreType.DMA for _ in range(2)],
            [pltpu.SemaphoreType.DMA for _ in range(2)],
        ],
    )
    def transform(imgs_hbm, luts_hbm, verts_hbm, out_hbm,
                  lut_v, verts_v, in_v, out_v, in_sem, out_sem):
        wid = lax.axis_index("s") * NC + lax.axis_index("c")
        b = wid // 2
        chunk0 = b * n_chunks_s + (wid % 2) * n_chunks
        pltpu.sync_copy(luts_hbm.at[pl.ds(b * LUT_PAD, LUT_PAD)], lut_v)
        pltpu.sync_copy(verts_hbm.at[pl.ds(b * 128, 128)], verts_v)

        def start_in(p, g):
            pltpu.async_copy(imgs_hbm.at[pl.ds((chunk0 + g) * CF, CF)],
                             in_v[p], in_sem[p])

        def wait_in(p):
            pltpu.make_async_copy(imgs_hbm.at[pl.ds(0, CF)], in_v[p],
                                  in_sem[p]).wait()

        def start_out(p, g):
            pltpu.async_copy(out_v[p], out_hbm.at[pl.ds((chunk0 + g) * CF, CF)],
                             out_sem[p])

        def wait_out(p):
            pltpu.make_async_copy(out_v[p], out_hbm.at[pl.ds(0, CF)],
                                  out_sem[p]).wait()

        def compute(p):
            def do_vec(i, carry2):
                off = i * L
                idxs = []
                fracs = []
                for c in range(3):
                    val = in_v[p][pl.ds(c * CHUNK + off, L)]
                    cbase = jnp.full((L,), c * V, jnp.int32)
                    lo = jnp.zeros((L,), jnp.int32)
                    for s in (16, 8, 4, 2, 1):
                        probe = lo + s
                        vv = plsc.load_gather(verts_v, [probe + cbase])
                        lo = jnp.where(vv <= val, probe, lo)
                    lov = plsc.load_gather(verts_v, [lo + cbase])
                    hiv = plsc.load_gather(verts_v, [lo + cbase + 1])
                    fr = jnp.clip((val - lov) / jnp.maximum(hiv - lov, 1e-10),
                                  0.0, 1.0)
                    idxs.append(lo)
                    fracs.append(fr)
                ir, ig, ib = idxs
                fr, fg, fb = fracs
                base = ib * V2 + ig * V + ir
                wr0 = 1.0 - fr
                wg0 = 1.0 - fg
                wb0 = 1.0 - fb
                wbg00 = wb0 * wg0
                wbg01 = wb0 * fg
                wbg10 = fb * wg0
                wbg11 = fb * fg
                ws = (wbg00 * wr0, wbg00 * fr, wbg01 * wr0, wbg01 * fr,
                      wbg10 * wr0, wbg10 * fr, wbg11 * wr0, wbg11 * fr)
                offs = (0, 1, V, V + 1, V2, V2 + 1, V2 + V, V2 + V + 1)
                for c in range(3):
                    basec = base + c * V3
                    acc = ws[0] * plsc.load_gather(lut_v, [basec])
                    for k in range(1, 8):
                        acc = acc + ws[k] * plsc.load_gather(lut_v,
                                                             [basec + offs[k]])
                    out_v[p][pl.ds(c * CHUNK + off, L)] = acc
                return carry2

            lax.fori_loop(0, CHUNK // L, do_vec, 0, unroll=False)

        start_in(0, 0)
        start_in(1, 1)

        def do_pair(gg, carry):
            for p in range(2):
                g = 2 * gg + p
                wait_in(p)

                @pl.when(gg > 0)
                def _():
                    wait_out(p)

                compute(p)
                start_out(p, g)

                @pl.when(g + 2 < n_chunks)
                def _():
                    start_in(p, g + 2)

            return carry

        lax.fori_loop(0, n_chunks // 2, do_pair, 0, unroll=False)
        wait_out(0)
        wait_out(1)

    return transform


def kernel(imgs, w1, b1, g1, be1, w2, b2, g2, be2, w3, b3, g3, be3,
           w4, b4, g4, be4, w5, b5, Wg, bg, Wb, Wa, ba):
    B, _, H, W = imgs.shape
    HW = H * W
    x = jax.image.resize(imgs, (B, 3, 256, 256), method='bilinear')
    x = _inorm2(_lrelu2(_conv2(x, w1, b1, 2)), g1, be1)
    x = _inorm2(_lrelu2(_conv2(x, w2, b2, 2)), g2, be2)
    x = _inorm2(_lrelu2(_conv2(x, w3, b3, 2)), g3, be3)
    x = _inorm2(_lrelu2(_conv2(x, w4, b4, 2)), g4, be4)
    x = _lrelu2(_conv2(x, w5, b5, 2))
    x = x.reshape(B, 128, 2, 4, 2, 4).mean(axis=(3, 5))
    codes = x.reshape(B, -1)
    weights = codes @ Wg.T + bg
    luts = weights @ Wb.T
    luts = jnp.pad(luts, ((0, 0), (0, LUT_PAD - LUT_LEN)))
    intervals = (codes @ Wa.T + ba).reshape(B, 3, V - 1)
    intervals = jax.nn.softmax(intervals, axis=-1)
    vertices = jnp.pad(jnp.cumsum(intervals, axis=-1), ((0, 0), (0, 0), (1, 0)))
    verts = jnp.pad(vertices.reshape(B, 3 * V), ((0, 0), (0, 128 - 3 * V)))

    imgs_t = _build_interleave(B, H, W)(imgs)
    out = _build_transform(B, HW)(imgs_t, luts.reshape(-1), verts.reshape(-1))
    return _build_deinterleave(B, H, W)(out)


# speedup vs baseline: 579.0670x; 1.0028x over previous
import functools

import jax
import jax.numpy as jnp
from jax import lax
from jax.experimental import pallas as pl
from jax.experimental.pallas import tpu as pltpu
from jax.experimental.pallas import tpu_sc as plsc

V = 33
V2 = V * V
V3 = V * V * V
LUT_LEN = 3 * V3
LUT_PAD = LUT_LEN + 5
NC, NS, L = 2, 16, 16
NW = NC * NS
CHUNK = 1024


def _conv2(x, w, b, stride):
    y = lax.conv_general_dilated(x, w, (stride, stride), ((1, 1), (1, 1)),
                                 dimension_numbers=('NCHW', 'OIHW', 'NCHW'))
    return y + b[None, :, None, None]


def _inorm2(x, g, b, eps=1e-5):
    m = jnp.mean(x, axis=(2, 3), keepdims=True)
    v = jnp.var(x, axis=(2, 3), keepdims=True)
    return (x - m) / jnp.sqrt(v + eps) * g[None, :, None, None] + b[None, :, None, None]


def _lrelu2(x):
    return jnp.where(x >= 0, x, 0.2 * x)


_NSPLIT = 8


@functools.cache
def _build_interleave(B, H, W):
    HW = H * W
    rpc = CHUNK // W
    rows = H // _NSPLIT
    ckb = rows // rpc
    FB = 3 * rows * W

    def body(x_ref, o_ref):
        for c in range(3):
            for k in range(ckb):
                for r in range(rpc):
                    o_ref[pl.ds(k * 3 * CHUNK + c * CHUNK + r * W, W)] = (
                        x_ref[0, c, k * rpc + r, :])

    return pl.pallas_call(
        body,
        grid=(B, _NSPLIT),
        in_specs=[pl.BlockSpec((1, 3, rows, W), lambda b, s: (b, 0, s, 0))],
        out_specs=pl.BlockSpec((FB,), lambda b, s: (b * _NSPLIT + s,)),
        out_shape=jax.ShapeDtypeStruct((B * 3 * HW,), jnp.float32),
    )


@functools.cache
def _build_deinterleave(B, H, W):
    HW = H * W
    rpc = CHUNK // W
    rows = H // _NSPLIT
    ckb = rows // rpc
    FB = 3 * rows * W

    def body(x_ref, o_ref):
        for c in range(3):
            for k in range(ckb):
                for r in range(rpc):
                    o_ref[0, c, k * rpc + r, :] = (
                        x_ref[pl.ds(k * 3 * CHUNK + c * CHUNK + r * W, W)])

    return pl.pallas_call(
        body,
        grid=(B, _NSPLIT),
        in_specs=[pl.BlockSpec((FB,), lambda b, s: (b * _NSPLIT + s,))],
        out_specs=pl.BlockSpec((1, 3, rows, W), lambda b, s: (b, 0, s, 0)),
        out_shape=jax.ShapeDtypeStruct((B, 3, H, W), jnp.float32),
    )


@functools.cache
def _build_transform(B, HW):
    half_px = HW // 2
    n_chunks = half_px // CHUNK
    n_chunks_s = HW // CHUNK
    CF = 3 * CHUNK
    mesh = plsc.VectorSubcoreMesh(core_axis_name="c", subcore_axis_name="s",
                                  num_cores=NC, num_subcores=NS)

    @functools.partial(
        pl.kernel,
        out_type=jax.ShapeDtypeStruct((B * 3 * HW,), jnp.float32),
        mesh=mesh,
        compiler_params=pltpu.CompilerParams(needs_layout_passes=False),
        scratch_types=[
            pltpu.VMEM((LUT_PAD,), jnp.float32),
            pltpu.VMEM((128,), jnp.float32),
            [pltpu.VMEM((CF,), jnp.float32) for _ in range(2)],
            [pltpu.VMEM((CF,), jnp.float32) for _ in range(2)],
            [pltpu.SemaphoreType.DMA for _ in range(2)],
            [pltpu.SemaphoreType.DMA for _ in range(2)],
        ],
    )
    def transform(imgs_hbm, luts_hbm, verts_hbm, out_hbm,
                  lut_v, verts_v, in_v, out_v, in_sem, out_sem):
        wid = lax.axis_index("s") * NC + lax.axis_index("c")
        b = wid // 2
        chunk0 = b * n_chunks_s + (wid % 2) * n_chunks
        pltpu.sync_copy(luts_hbm.at[pl.ds(b * LUT_PAD, LUT_PAD)], lut_v)
        pltpu.sync_copy(verts_hbm.at[pl.ds(b * 128, 128)], verts_v)

        def start_in(p, g):
            pltpu.async_copy(imgs_hbm.at[pl.ds((chunk0 + g) * CF, CF)],
                             in_v[p], in_sem[p])

        def wait_in(p):
            pltpu.make_async_copy(imgs_hbm.at[pl.ds(0, CF)], in_v[p],
                                  in_sem[p]).wait()

        def start_out(p, g):
            pltpu.async_copy(out_v[p], out_hbm.at[pl.ds((chunk0 + g) * CF, CF)],
                             out_sem[p])

        def wait_out(p):
            pltpu.make_async_copy(out_v[p], out_hbm.at[pl.ds(0, CF)],
                                  out_sem[p]).wait()

        def compute(p):
            def do_vec(i, carry2):
                off = i * L
                idxs = []
                fracs = []
                for c in range(3):
                    val = in_v[p][pl.ds(c * CHUNK + off, L)]
                    cbase = jnp.full((L,), c * V, jnp.int32)
                    lo = jnp.zeros((L,), jnp.int32)
                    for s in (16, 8, 4, 2, 1):
                        probe = lo + s
                        vv = plsc.load_gather(verts_v, [probe + cbase])
                        lo = jnp.where(vv <= val, probe, lo)
                    lov = plsc.load_gather(verts_v, [lo + cbase])
                    hiv = plsc.load_gather(verts_v, [lo + cbase + 1])
                    fr = jnp.clip((val - lov) / jnp.maximum(hiv - lov, 1e-10),
                                  0.0, 1.0)
                    idxs.append(lo)
                    fracs.append(fr)
                ir, ig, ib = idxs
                fr, fg, fb = fracs
                base = ib * V2 + ig * V + ir
                wr0 = 1.0 - fr
                wg0 = 1.0 - fg
                wb0 = 1.0 - fb
                wbg00 = wb0 * wg0
                wbg01 = wb0 * fg
                wbg10 = fb * wg0
                wbg11 = fb * fg
                ws = (wbg00 * wr0, wbg00 * fr, wbg01 * wr0, wbg01 * fr,
                      wbg10 * wr0, wbg10 * fr, wbg11 * wr0, wbg11 * fr)
                offs = (0, 1, V, V + 1, V2, V2 + 1, V2 + V, V2 + V + 1)
                for c in range(3):
                    basec = base + c * V3
                    acc = ws[0] * plsc.load_gather(lut_v, [basec])
                    for k in range(1, 8):
                        acc = acc + ws[k] * plsc.load_gather(lut_v,
                                                             [basec + offs[k]])
                    out_v[p][pl.ds(c * CHUNK + off, L)] = acc
                return carry2

            lax.fori_loop(0, CHUNK // L, do_vec, 0, unroll=False)

        start_in(0, 0)
        start_in(1, 1)

        def do_pair(gg, carry):
            for p in range(2):
                g = 2 * gg + p
                wait_in(p)

                @pl.when(gg > 0)
                def _():
                    wait_out(p)

                compute(p)
                start_out(p, g)

                @pl.when(g + 2 < n_chunks)
                def _():
                    start_in(p, g + 2)

            return carry

        lax.fori_loop(0, n_chunks // 2, do_pair, 0, unroll=False)
        wait_out(0)
        wait_out(1)

    return transform


def kernel(imgs, w1, b1, g1, be1, w2, b2, g2, be2, w3, b3, g3, be3,
           w4, b4, g4, be4, w5, b5, Wg, bg, Wb, Wa, ba):
    B, _, H, W = imgs.shape
    HW = H * W
    Mh = jax.image.resize(jnp.eye(H, dtype=jnp.float32), (256, H),
                          method='bilinear')
    Mw = jax.image.resize(jnp.eye(W, dtype=jnp.float32), (256, W),
                          method='bilinear')
    x = jnp.einsum('oh,bchw->bcow', Mh, imgs,
                   precision=lax.Precision.HIGHEST)
    x = jnp.einsum('pw,bcow->bcop', Mw, x,
                   precision=lax.Precision.HIGHEST)
    x = _inorm2(_lrelu2(_conv2(x, w1, b1, 2)), g1, be1)
    x = _inorm2(_lrelu2(_conv2(x, w2, b2, 2)), g2, be2)
    x = _inorm2(_lrelu2(_conv2(x, w3, b3, 2)), g3, be3)
    x = _inorm2(_lrelu2(_conv2(x, w4, b4, 2)), g4, be4)
    x = _lrelu2(_conv2(x, w5, b5, 2))
    x = x.reshape(B, 128, 2, 4, 2, 4).mean(axis=(3, 5))
    codes = x.reshape(B, -1)
    weights = codes @ Wg.T + bg
    luts = weights @ Wb.T
    luts = jnp.pad(luts, ((0, 0), (0, LUT_PAD - LUT_LEN)))
    intervals = (codes @ Wa.T + ba).reshape(B, 3, V - 1)
    intervals = jax.nn.softmax(intervals, axis=-1)
    vertices = jnp.pad(jnp.cumsum(intervals, axis=-1), ((0, 0), (0, 0), (1, 0)))
    verts = jnp.pad(vertices.reshape(B, 3 * V), ((0, 0), (0, 128 - 3 * V)))

    imgs_t = _build_interleave(B, H, W)(imgs)
    out = _build_transform(B, HW)(imgs_t, luts.reshape(-1), verts.reshape(-1))
    return _build_deinterleave(B, H, W)(out)


# speedup vs baseline: 1392.7926x; 2.4052x over previous
import functools

import jax
import jax.numpy as jnp
from jax import lax
from jax.experimental import pallas as pl
from jax.experimental.pallas import tpu as pltpu
from jax.experimental.pallas import tpu_sc as plsc

V = 33
V2 = V * V
V3 = V * V * V
LUT_LEN = 3 * V3
LUT_PAD = LUT_LEN + 5
NC, NS, L = 2, 16, 16
NW = NC * NS
CHUNK = 1024


def _conv2(x, w, b, stride):
    y = lax.conv_general_dilated(x, w, (stride, stride), ((1, 1), (1, 1)),
                                 dimension_numbers=('HWCN', 'OIHW', 'HWCN'))
    return y + b[None, None, :, None]


def _inorm2(x, g, b, eps=1e-5):
    m = jnp.mean(x, axis=(0, 1), keepdims=True)
    v = jnp.var(x, axis=(0, 1), keepdims=True)
    return (x - m) / jnp.sqrt(v + eps) * g[None, None, :, None] + b[None, None, :, None]


def _lrelu2(x):
    return jnp.where(x >= 0, x, 0.2 * x)


_NSPLIT = 8


@functools.cache
def _build_interleave(B, H, W):
    HW = H * W
    rpc = CHUNK // W
    rows = H // _NSPLIT
    ckb = rows // rpc
    FB = 3 * rows * W

    def body(x_ref, o_ref, o2_ref):
        for c in range(3):
            for k in range(ckb):
                for r in range(rpc):
                    o_ref[pl.ds(k * 3 * CHUNK + c * CHUNK + r * W, W)] = (
                        x_ref[0, c, k * rpc + r, :])
        o2_ref[:, 0] = x_ref[0]

    return pl.pallas_call(
        body,
        grid=(B, _NSPLIT),
        in_specs=[pl.BlockSpec((1, 3, rows, W), lambda b, s: (b, 0, s, 0))],
        out_specs=[
            pl.BlockSpec((FB,), lambda b, s: (b * _NSPLIT + s,)),
            pl.BlockSpec((3, 1, rows, W), lambda b, s: (0, b, s, 0)),
        ],
        out_shape=[
            jax.ShapeDtypeStruct((B * 3 * HW,), jnp.float32),
            jax.ShapeDtypeStruct((3, B, H, W), jnp.float32),
        ],
    )


@functools.cache
def _build_deinterleave(B, H, W):
    HW = H * W
    rpc = CHUNK // W
    rows = H // _NSPLIT
    ckb = rows // rpc
    FB = 3 * rows * W

    def body(x_ref, o_ref):
        for c in range(3):
            for k in range(ckb):
                for r in range(rpc):
                    o_ref[0, c, k * rpc + r, :] = (
                        x_ref[pl.ds(k * 3 * CHUNK + c * CHUNK + r * W, W)])

    return pl.pallas_call(
        body,
        grid=(B, _NSPLIT),
        in_specs=[pl.BlockSpec((FB,), lambda b, s: (b * _NSPLIT + s,))],
        out_specs=pl.BlockSpec((1, 3, rows, W), lambda b, s: (b, 0, s, 0)),
        out_shape=jax.ShapeDtypeStruct((B, 3, H, W), jnp.float32),
    )


@functools.cache
def _build_transform(B, HW):
    half_px = HW // 2
    n_chunks = half_px // CHUNK
    n_chunks_s = HW // CHUNK
    CF = 3 * CHUNK
    mesh = plsc.VectorSubcoreMesh(core_axis_name="c", subcore_axis_name="s",
                                  num_cores=NC, num_subcores=NS)

    @functools.partial(
        pl.kernel,
        out_type=jax.ShapeDtypeStruct((B * 3 * HW,), jnp.float32),
        mesh=mesh,
        compiler_params=pltpu.CompilerParams(needs_layout_passes=False),
        scratch_types=[
            pltpu.VMEM((LUT_PAD,), jnp.float32),
            pltpu.VMEM((128,), jnp.float32),
            [pltpu.VMEM((CF,), jnp.float32) for _ in range(2)],
            [pltpu.VMEM((CF,), jnp.float32) for _ in range(2)],
            [pltpu.SemaphoreType.DMA for _ in range(2)],
            [pltpu.SemaphoreType.DMA for _ in range(2)],
        ],
    )
    def transform(imgs_hbm, luts_hbm, verts_hbm, out_hbm,
                  lut_v, verts_v, in_v, out_v, in_sem, out_sem):
        wid = lax.axis_index("s") * NC + lax.axis_index("c")
        b = wid // 2
        chunk0 = b * n_chunks_s + (wid % 2) * n_chunks
        pltpu.sync_copy(luts_hbm.at[pl.ds(b * LUT_PAD, LUT_PAD)], lut_v)
        pltpu.sync_copy(verts_hbm.at[pl.ds(b * 128, 128)], verts_v)

        def start_in(p, g):
            pltpu.async_copy(imgs_hbm.at[pl.ds((chunk0 + g) * CF, CF)],
                             in_v[p], in_sem[p])

        def wait_in(p):
            pltpu.make_async_copy(imgs_hbm.at[pl.ds(0, CF)], in_v[p],
                                  in_sem[p]).wait()

        def start_out(p, g):
            pltpu.async_copy(out_v[p], out_hbm.at[pl.ds((chunk0 + g) * CF, CF)],
                             out_sem[p])

        def wait_out(p):
            pltpu.make_async_copy(out_v[p], out_hbm.at[pl.ds(0, CF)],
                                  out_sem[p]).wait()

        def compute(p):
            def do_vec(i, carry2):
                off = i * L
                idxs = []
                fracs = []
                for c in range(3):
                    val = in_v[p][pl.ds(c * CHUNK + off, L)]
                    cbase = jnp.full((L,), c * V, jnp.int32)
                    lo = jnp.zeros((L,), jnp.int32)
                    for s in (16, 8, 4, 2, 1):
                        probe = lo + s
                        vv = plsc.load_gather(verts_v, [probe + cbase])
                        lo = jnp.where(vv <= val, probe, lo)
                    lov = plsc.load_gather(verts_v, [lo + cbase])
                    hiv = plsc.load_gather(verts_v, [lo + cbase + 1])
                    fr = jnp.clip((val - lov) / jnp.maximum(hiv - lov, 1e-10),
                                  0.0, 1.0)
                    idxs.append(lo)
                    fracs.append(fr)
                ir, ig, ib = idxs
                fr, fg, fb = fracs
                base = ib * V2 + ig * V + ir
                wr0 = 1.0 - fr
                wg0 = 1.0 - fg
                wb0 = 1.0 - fb
                wbg00 = wb0 * wg0
                wbg01 = wb0 * fg
                wbg10 = fb * wg0
                wbg11 = fb * fg
                ws = (wbg00 * wr0, wbg00 * fr, wbg01 * wr0, wbg01 * fr,
                      wbg10 * wr0, wbg10 * fr, wbg11 * wr0, wbg11 * fr)
                offs = (0, 1, V, V + 1, V2, V2 + 1, V2 + V, V2 + V + 1)
                for c in range(3):
                    basec = base + c * V3
                    acc = ws[0] * plsc.load_gather(lut_v, [basec])
                    for k in range(1, 8):
                        acc = acc + ws[k] * plsc.load_gather(lut_v,
                                                             [basec + offs[k]])
                    out_v[p][pl.ds(c * CHUNK + off, L)] = acc
                return carry2

            lax.fori_loop(0, CHUNK // L, do_vec, 0, unroll=False)

        start_in(0, 0)
        start_in(1, 1)

        def do_pair(gg, carry):
            for p in range(2):
                g = 2 * gg + p
                wait_in(p)

                @pl.when(gg > 0)
                def _():
                    wait_out(p)

                compute(p)
                start_out(p, g)

                @pl.when(g + 2 < n_chunks)
                def _():
                    start_in(p, g + 2)

            return carry

        lax.fori_loop(0, n_chunks // 2, do_pair, 0, unroll=False)
        wait_out(0)
        wait_out(1)

    return transform


def kernel(imgs, w1, b1, g1, be1, w2, b2, g2, be2, w3, b3, g3, be3,
           w4, b4, g4, be4, w5, b5, Wg, bg, Wb, Wa, ba):
    B, _, H, W = imgs.shape
    HW = H * W
    imgs_t, imgs_cb = _build_interleave(B, H, W)(imgs)
    Mh = jax.image.resize(jnp.eye(H, dtype=jnp.float32), (256, H),
                          method='bilinear')
    Mw = jax.image.resize(jnp.eye(W, dtype=jnp.float32), (256, W),
                          method='bilinear')
    x = jnp.einsum('pw,cbhw->pcbh', Mw, imgs_cb,
                   precision=lax.Precision.HIGHEST)
    x = jnp.einsum('oh,pcbh->opcb', Mh, x,
                   precision=lax.Precision.HIGHEST)
    x = _inorm2(_lrelu2(_conv2(x, w1, b1, 2)), g1, be1)
    x = _inorm2(_lrelu2(_conv2(x, w2, b2, 2)), g2, be2)
    x = _inorm2(_lrelu2(_conv2(x, w3, b3, 2)), g3, be3)
    x = _inorm2(_lrelu2(_conv2(x, w4, b4, 2)), g4, be4)
    x = _lrelu2(_conv2(x, w5, b5, 2))
    x = x.reshape(2, 4, 2, 4, 128, B).mean(axis=(1, 3))
    codes = x.transpose(3, 2, 0, 1).reshape(B, -1)
    weights = codes @ Wg.T + bg
    luts = weights @ Wb.T
    luts = jnp.pad(luts, ((0, 0), (0, LUT_PAD - LUT_LEN)))
    intervals = (codes @ Wa.T + ba).reshape(B, 3, V - 1)
    intervals = jax.nn.softmax(intervals, axis=-1)
    vertices = jnp.pad(jnp.cumsum(intervals, axis=-1), ((0, 0), (0, 0), (1, 0)))
    verts = jnp.pad(vertices.reshape(B, 3 * V), ((0, 0), (0, 128 - 3 * V)))

    out = _build_transform(B, HW)(imgs_t, luts.reshape(-1), verts.reshape(-1))
    return _build_deinterleave(B, H, W)(out)
